# baseline (device time: 24461 ns/iter reference)
import jax
import jax.numpy as jnp
from jax import lax
from jax.experimental import pallas as pl
from jax.experimental.pallas import tpu as pltpu

N_DEV = 4


def kernel(x, Wq, K_ext, V_ext, Wo):
    B, Sq, Din = x.shape
    _, Skv_l, Hq, Dh = K_ext.shape
    HD = Hq * Dh
    Dout = Wo.shape[1]
    BLK = 64
    ROWS = Sq + 8

    K2 = K_ext.reshape(B, Skv_l, HD)
    V2 = V_ext.reshape(B, Skv_l, HD)

    def body(x_ref, wq_ref, k_ref, v_ref, wo_ref, out_ref,
             comm_ref, acc_ref, send_sems, recv_sems):
        my = lax.axis_index("i")

        barrier_sem = pltpu.get_barrier_semaphore()
        for o in (1, 2, 3):
            pl.semaphore_signal(
                barrier_sem, inc=1,
                device_id=(lax.rem(my + o, N_DEV),),
                device_id_type=pl.DeviceIdType.MESH,
            )
        pl.semaphore_wait(barrier_sem, N_DEV - 1)

        blk = (
            lax.broadcasted_iota(jnp.int32, (Sq, Skv_l), 0) // BLK
            == lax.broadcasted_iota(jnp.int32, (Sq, Skv_l), 1) // BLK
        )

        for b in range(B):
            qb = jnp.dot(x_ref[b, :, :], wq_ref[:, :],
                         preferred_element_type=jnp.float32)
            for h in range(Hq):
                qh = qb[:, h * Dh:(h + 1) * Dh]
                kh = k_ref[b, :, h * Dh:(h + 1) * Dh]
                s = lax.dot_general(
                    qh, kh, (((1,), (1,)), ((), ())),
                    preferred_element_type=jnp.float32) * 0.125
                w = jnp.where(blk, jnp.exp(s), 0.0)
                ctx = jnp.dot(w, v_ref[b, :, h * Dh:(h + 1) * Dh],
                              preferred_element_type=jnp.float32)
                comm_ref[0, b, 0:Sq, h * Dh:(h + 1) * Dh] = ctx
                comm_ref[0, b, Sq + h:Sq + h + 1, :] = (
                    jnp.sum(w, axis=1)[None, :])
            comm_ref[0, b, Sq + Hq:ROWS, :] = jnp.zeros(
                (ROWS - Sq - Hq, HD), jnp.float32)

        acc_ref[...] = comm_ref[0]

        rdmas = []
        for o in (1, 2, 3):
            rdma = pltpu.make_async_remote_copy(
                src_ref=comm_ref.at[0],
                dst_ref=comm_ref.at[N_DEV - o],
                send_sem=send_sems.at[o - 1],
                recv_sem=recv_sems.at[N_DEV - o - 1],
                device_id=(lax.rem(my + o, N_DEV),),
                device_id_type=pl.DeviceIdType.MESH,
            )
            rdma.start()
            rdmas.append(rdma)

        for o in (3, 2, 1):
            rdmas[o - 1].wait_recv()
            acc_ref[...] += comm_ref[N_DEV - o]
        for o in (1, 2, 3):
            rdmas[o - 1].wait_send()

        for b in range(B):
            parts = []
            for h in range(Hq):
                l = acc_ref[b, Sq + h, :]
                ctx = acc_ref[b, 0:Sq, h * Dh:(h + 1) * Dh]
                parts.append(ctx / l[:, None])
            norm = jnp.concatenate(parts, axis=1)
            out_ref[b, :, :] = jnp.dot(norm, wo_ref[:, :],
                                       preferred_element_type=jnp.float32)

    return pl.pallas_call(
        body,
        out_shape=jax.ShapeDtypeStruct((B, Sq, Dout), jnp.float32),
        in_specs=[pl.BlockSpec(memory_space=pltpu.VMEM)] * 5,
        out_specs=pl.BlockSpec(memory_space=pltpu.VMEM),
        scratch_shapes=[
            pltpu.VMEM((N_DEV, B, ROWS, HD), jnp.float32),
            pltpu.VMEM((B, ROWS, HD), jnp.float32),
            pltpu.SemaphoreType.DMA((N_DEV - 1,)),
            pltpu.SemaphoreType.DMA((N_DEV - 1,)),
        ],
        compiler_params=pltpu.CompilerParams(collective_id=0),
    )(x, Wq, K2, V2, Wo)


# device time: 19375 ns/iter; 1.2625x vs baseline; 1.2625x over previous
import jax
import jax.numpy as jnp
from jax import lax
from jax.experimental import pallas as pl
from jax.experimental.pallas import tpu as pltpu

N_DEV = 4


def kernel(x, Wq, K_ext, V_ext, Wo):
    B, Sq, Din = x.shape
    _, Skv_l, Hq, Dh = K_ext.shape
    HD = Hq * Dh
    Dout = Wo.shape[1]
    BLK = 64
    ROWS = Sq + 16

    K2 = K_ext.reshape(B, Skv_l, HD)
    V2 = V_ext.reshape(B, Skv_l, HD)

    def body(x_ref, wq_ref, k_ref, v_ref, wo_ref, out_ref,
             comm_ref, send_sems, recv_sems):
        my = lax.axis_index("i")
        p1 = jnp.bitwise_xor(my, 1)
        p2 = jnp.bitwise_xor(my, 2)

        barrier_sem = pltpu.get_barrier_semaphore()
        for p in (p1, p2):
            pl.semaphore_signal(
                barrier_sem, inc=1,
                device_id=(p,), device_id_type=pl.DeviceIdType.MESH,
            )
        pl.semaphore_wait(barrier_sem, 2)

        blk = (
            lax.broadcasted_iota(jnp.int32, (Sq, Skv_l), 0) // BLK
            == lax.broadcasted_iota(jnp.int32, (Sq, Skv_l), 1) // BLK
        )

        for b in range(B):
            qb = jnp.dot(x_ref[b, :, :], wq_ref[:, :],
                         preferred_element_type=jnp.float32)
            for h in range(Hq):
                qh = qb[:, h * Dh:(h + 1) * Dh]
                kh = k_ref[b, :, h * Dh:(h + 1) * Dh]
                s = lax.dot_general(
                    qh, kh, (((1,), (1,)), ((), ())),
                    preferred_element_type=jnp.float32) * 0.125
                w = jnp.where(blk, jnp.exp(s), 0.0)
                ctx = jnp.dot(w, v_ref[b, :, h * Dh:(h + 1) * Dh],
                              preferred_element_type=jnp.float32)
                comm_ref[0, b, 0:Sq, h * Dh:(h + 1) * Dh] = (
                    ctx.astype(jnp.bfloat16))
                comm_ref[0, b, Sq + h:Sq + h + 1, :] = (
                    jnp.sum(w, axis=1)[None, :].astype(jnp.bfloat16))
            comm_ref[0, b, Sq + Hq:ROWS, :] = jnp.zeros(
                (ROWS - Sq - Hq, HD), jnp.bfloat16)

        rdma1 = pltpu.make_async_remote_copy(
            src_ref=comm_ref.at[0],
            dst_ref=comm_ref.at[1],
            send_sem=send_sems.at[0],
            recv_sem=recv_sems.at[0],
            device_id=(p1,),
            device_id_type=pl.DeviceIdType.MESH,
        )
        rdma1.start()
        rdma1.wait_recv()
        rdma1.wait_send()
        comm_ref[0] = (comm_ref[0] + comm_ref[1]).astype(jnp.bfloat16)

        rdma2 = pltpu.make_async_remote_copy(
            src_ref=comm_ref.at[0],
            dst_ref=comm_ref.at[2],
            send_sem=send_sems.at[1],
            recv_sem=recv_sems.at[1],
            device_id=(p2,),
            device_id_type=pl.DeviceIdType.MESH,
        )
        rdma2.start()
        rdma2.wait_recv()

        for b in range(B):
            parts = []
            for h in range(Hq):
                l = (comm_ref[0, b, Sq + h, :].astype(jnp.float32)
                     + comm_ref[2, b, Sq + h, :].astype(jnp.float32))
                ctx = (comm_ref[0, b, 0:Sq, h * Dh:(h + 1) * Dh]
                       .astype(jnp.float32)
                       + comm_ref[2, b, 0:Sq, h * Dh:(h + 1) * Dh]
                       .astype(jnp.float32))
                parts.append(ctx / l[:, None])
            norm = jnp.concatenate(parts, axis=1)
            out_ref[b, :, :] = jnp.dot(norm, wo_ref[:, :],
                                       preferred_element_type=jnp.float32)

        rdma2.wait_send()

    return pl.pallas_call(
        body,
        out_shape=jax.ShapeDtypeStruct((B, Sq, Dout), jnp.float32),
        in_specs=[pl.BlockSpec(memory_space=pltpu.VMEM)] * 5,
        out_specs=pl.BlockSpec(memory_space=pltpu.VMEM),
        scratch_shapes=[
            pltpu.VMEM((3, B, ROWS, HD), jnp.bfloat16),
            pltpu.SemaphoreType.DMA((2,)),
            pltpu.SemaphoreType.DMA((2,)),
        ],
        compiler_params=pltpu.CompilerParams(collective_id=0),
    )(x, Wq, K2, V2, Wo)
